# baseline (speedup 1.0000x reference)
import functools

import jax
import jax.numpy as jnp
from jax import lax
from jax.experimental import pallas as pl
from jax.experimental.pallas import tpu as pltpu
from jax.experimental.pallas import tpu_sc as plsc

try:
    _info = plsc.get_sparse_core_info()
    NC, NS = _info.num_cores, _info.num_subcores
except Exception:
    NC, NS = 2, 16
NW = NC * NS

C = 128
BPT = 512

_params = pltpu.CompilerParams(use_tc_tiling_on_sc=True, needs_layout_passes=False)


@functools.cache
def _transpose(V, D):
    mesh = plsc.VectorSubcoreMesh(
        core_axis_name="c", subcore_axis_name="s", num_cores=NC, num_subcores=NS
    )
    nfull = V // C
    tail = V - nfull * C
    per_w = nfull // NW
    extra = nfull - per_w * NW

    @functools.partial(
        pl.kernel,
        mesh=mesh,
        out_type=jax.ShapeDtypeStruct((V // 2, 2 * D), jnp.float32),
        scratch_types=[
            *[pltpu.VMEM((D, C), jnp.float32) for _ in range(2)],
            *[pltpu.VMEM((C // 2, 2 * D), jnp.float32) for _ in range(2)],
            pltpu.VMEM((D, D), jnp.float32),
            *[pltpu.SemaphoreType.DMA for _ in range(4)],
        ],
        compiler_params=_params,
    )
    def tk(tabT_hbm, out_hbm, *rest):
        sb = rest[0:2]
        db = rest[2:4]
        sbt = rest[4]
        sem_i = rest[5:7]
        sem_o = rest[7:9]
        wid = lax.axis_index("s") * NC + lax.axis_index("c")
        start = wid * per_w + jnp.minimum(wid, extra)
        iota = lax.iota(jnp.int32, 16)

        def load(i, s):
            pltpu.async_copy(
                tabT_hbm.at[:, pl.ds(pl.multiple_of((start + i) * C, C), C)], sb[s], sem_i[s]
            )

        def wait_load(i, s):
            pltpu.make_async_copy(
                tabT_hbm.at[:, pl.ds(pl.multiple_of((start + i) * C, C), C)], sb[s], sem_i[s]
            ).wait()

        def store(i, s):
            pltpu.async_copy(
                db[s], out_hbm.at[pl.ds(pl.multiple_of((start + i) * (C // 2), C // 2), C // 2)], sem_o[s]
            )

        def wait_store(i, s):
            pltpu.make_async_copy(
                db[s], out_hbm.at[pl.ds(pl.multiple_of((start + i) * (C // 2), C // 2), C // 2)], sem_o[s]
            ).wait()

        def transpose_block(src, dst, col_off):
            def body(k2, carry):
                for gj in range(2 * D // 16):
                    rows = iota + (gj * 16) % D
                    col = col_off + 2 * k2 + (gj * 16) // D
                    cols = jnp.full((16,), 0, jnp.int32) + col
                    dst[k2, pl.ds(gj * 16, 16)] = plsc.load_gather(src, [rows, cols])
                return carry

            lax.fori_loop(0, C // 2, body, 0, unroll=False)

        load(0, 0)

        def pair(i2, carry):
            for b in range(2):
                i = i2 * 2 + b
                s = b
                wait_load(i, s)

                @pl.when(i + 1 < per_w)
                def _():
                    load(i + 1, 1 - s)

                @pl.when(i >= 2)
                def _():
                    wait_store(i - 2, s)

                transpose_block(sb[s], db[s], 0)
                store(i, s)
            return carry

        lax.fori_loop(0, per_w // 2, pair, 0, unroll=False)
        wait_store(per_w - 2, 0)
        wait_store(per_w - 1, 1)

        @pl.when(wid < extra)
        def _():
            i = per_w
            load(i, 0)
            wait_load(i, 0)
            transpose_block(sb[0], db[0], 0)
            store(i, 0)
            wait_store(i, 0)

        if tail:
            @pl.when(wid == NW - 1)
            def _():
                pltpu.sync_copy(tabT_hbm.at[:, pl.ds(V - tail, tail)], sbt)

                def tbody(k2, carry):
                    for gj in range(2 * D // 16):
                        rows = iota + (gj * 16) % D
                        col = 2 * k2 + (gj * 16) // D
                        cols = jnp.full((16,), 0, jnp.int32) + col
                        db[1][k2, pl.ds(gj * 16, 16)] = plsc.load_gather(
                            sbt, [rows, cols]
                        )
                    return carry

                lax.fori_loop(0, tail // 2, tbody, 0, unroll=False)
                pltpu.sync_copy(
                    db[1].at[pl.ds(0, tail // 2)],
                    out_hbm.at[pl.ds((V - tail) // 2, tail // 2)],
                )

    return tk


@functools.cache
def _gather(B, H, D):
    mesh = plsc.VectorSubcoreMesh(
        core_axis_name="c", subcore_axis_name="s", num_cores=NC, num_subcores=NS
    )
    nblk = BPT // C

    @functools.partial(
        pl.kernel,
        mesh=mesh,
        out_type=jax.ShapeDtypeStruct((H, D, B), jnp.float32),
        scratch_types=[
            pltpu.VMEM((H, BPT), jnp.int32),
            pltpu.VMEM((nblk, C), jnp.int32),
            pltpu.VMEM((nblk, C), jnp.int32),
            *[pltpu.VMEM((C, 2 * D), jnp.float32) for _ in range(nblk)],
            *[pltpu.VMEM((D, C), jnp.float32) for _ in range(2)],
            *[pltpu.SemaphoreType.DMA for _ in range(nblk)],
            *[pltpu.SemaphoreType.DMA for _ in range(2)],
        ],
        compiler_params=_params,
    )
    def gk(idxT_hbm, tabP_hbm, out_hbm, idx_v, vp_v, hf_v, *rest):
        bufs = rest[0:nblk]
        bufT = rest[nblk : nblk + 2]
        sem_g = rest[nblk + 2 : 2 * nblk + 2]
        sem_s = rest[2 * nblk + 2 : 2 * nblk + 4]
        wid = lax.axis_index("s") * NC + lax.axis_index("c")
        bs = wid * BPT
        pltpu.sync_copy(idxT_hbm.at[:, pl.ds(pl.multiple_of(bs, BPT), BPT)], idx_v)
        iota = lax.iota(jnp.int32, 16)

        def prep(h, blk):
            for g in range(C // 16):
                seg = idx_v[h, pl.ds(blk * C + g * 16, 16)]
                vp_v[blk, pl.ds(g * 16, 16)] = seg >> 1
                hf_v[blk, pl.ds(g * 16, 16)] = (seg & 1) * D

        def fire_gather(blk):
            pltpu.async_copy(tabP_hbm.at[vp_v.at[blk]], bufs[blk], sem_g[blk])

        def wait_gather(blk):
            pltpu.make_async_copy(
                tabP_hbm.at[vp_v.at[blk]], bufs[blk], sem_g[blk]
            ).wait()

        def fire_store(h, blk):
            pltpu.async_copy(
                bufT[blk % 2],
                out_hbm.at[h, :, pl.ds(pl.multiple_of(bs + blk * C, C), C)],
                sem_s[blk % 2],
            )

        def wait_store(h, blk):
            pltpu.make_async_copy(
                bufT[blk % 2],
                out_hbm.at[h, :, pl.ds(pl.multiple_of(bs + blk * C, C), C)],
                sem_s[blk % 2],
            ).wait()

        def transpose_chunk(blk):
            src = bufs[blk]
            dst = bufT[blk % 2]

            def body(d, carry):
                for g in range(C // 16):
                    rows = iota + g * 16
                    cols = hf_v[blk, pl.ds(g * 16, 16)] + d
                    dst[d, pl.ds(g * 16, 16)] = plsc.load_gather(src, [rows, cols])
                return carry

            lax.fori_loop(0, D, body, 0, unroll=False)

        def chunk(h, blk, do_fire, do_wait_store):
            if do_fire:
                h2 = h + (blk + 2) // nblk
                blk2 = (blk + 2) % nblk
                prep(h2, blk2)
                fire_gather(blk2)
            wait_gather(blk)
            if do_wait_store:
                wait_store(h, blk)
            transpose_chunk(blk)
            fire_store(h, blk)

        prep(0, 0)
        fire_gather(0)
        prep(0, 1)
        fire_gather(1)
        for blk in range(nblk):
            chunk(0, blk, True, blk >= 2)

        def do_h(h, carry):
            for blk in range(nblk):
                chunk(h, blk, True, True)
            return carry

        lax.fori_loop(1, H - 1, do_h, 0, unroll=False)
        for blk in range(nblk):
            chunk(H - 1, blk, blk < 2, True)
        wait_store(H - 1, 2)
        wait_store(H - 1, 3)

    return gk


def kernel(input_variable, embedding_weight):
    B, H = input_variable.shape
    V, D = embedding_weight.shape
    idxT = input_variable.astype(jnp.int32).T
    tabT = embedding_weight.T
    tabP = _transpose(V, D)(tabT)
    out = _gather(B, H, D)(idxT, tabP)
    return jnp.transpose(out, (2, 0, 1))


# speedup vs baseline: 1.2542x; 1.2542x over previous
import functools

import jax
import jax.numpy as jnp
from jax import lax
from jax.experimental import pallas as pl
from jax.experimental.pallas import tpu as pltpu
from jax.experimental.pallas import tpu_sc as plsc

try:
    _info = plsc.get_sparse_core_info()
    NC, NS = _info.num_cores, _info.num_subcores
except Exception:
    NC, NS = 2, 16
NW = NC * NS

C = 128
BPT = 512

_params = pltpu.CompilerParams(use_tc_tiling_on_sc=True, needs_layout_passes=False)


@functools.cache
def _transpose(V, D):
    mesh = plsc.VectorSubcoreMesh(
        core_axis_name="c", subcore_axis_name="s", num_cores=NC, num_subcores=NS
    )
    nfull = V // C
    tail = V - nfull * C
    per_w = nfull // NW
    extra = nfull - per_w * NW

    @functools.partial(
        pl.kernel,
        mesh=mesh,
        out_type=jax.ShapeDtypeStruct((V // 2, 2 * D), jnp.float32),
        scratch_types=[
            *[pltpu.VMEM((D, C), jnp.float32) for _ in range(2)],
            *[pltpu.VMEM((C // 2, 2 * D), jnp.float32) for _ in range(2)],
            pltpu.VMEM((D, D), jnp.float32),
            *[pltpu.SemaphoreType.DMA for _ in range(4)],
        ],
        compiler_params=_params,
    )
    def tk(tabT_hbm, out_hbm, *rest):
        sb = rest[0:2]
        db = rest[2:4]
        sbt = rest[4]
        sem_i = rest[5:7]
        sem_o = rest[7:9]
        wid = lax.axis_index("s") * NC + lax.axis_index("c")
        start = wid * per_w + jnp.minimum(wid, extra)
        iota = lax.iota(jnp.int32, 16)

        def load(i, s):
            pltpu.async_copy(
                tabT_hbm.at[:, pl.ds(pl.multiple_of((start + i) * C, C), C)], sb[s], sem_i[s]
            )

        def wait_load(i, s):
            pltpu.make_async_copy(
                tabT_hbm.at[:, pl.ds(pl.multiple_of((start + i) * C, C), C)], sb[s], sem_i[s]
            ).wait()

        def store(i, s):
            pltpu.async_copy(
                db[s], out_hbm.at[pl.ds(pl.multiple_of((start + i) * (C // 2), C // 2), C // 2)], sem_o[s]
            )

        def wait_store(i, s):
            pltpu.make_async_copy(
                db[s], out_hbm.at[pl.ds(pl.multiple_of((start + i) * (C // 2), C // 2), C // 2)], sem_o[s]
            ).wait()

        rows_l = tuple(iota + (gj * 16) % D for gj in range(2 * D // 16))
        zeros16 = jnp.full((16,), 0, jnp.int32)

        def transpose_block(src, dst, col_off):
            def body(k2, carry):
                base = zeros16 + (col_off + 2 * k2)
                for gj in range(2 * D // 16):
                    cols = base + (gj * 16) // D
                    dst[k2, pl.ds(gj * 16, 16)] = plsc.load_gather(
                        src, [rows_l[gj], cols]
                    )
                return carry

            lax.fori_loop(0, C // 2, body, 0, unroll=4)

        load(0, 0)

        def pair(i2, carry):
            for b in range(2):
                i = i2 * 2 + b
                s = b
                wait_load(i, s)

                @pl.when(i + 1 < per_w)
                def _():
                    load(i + 1, 1 - s)

                @pl.when(i >= 2)
                def _():
                    wait_store(i - 2, s)

                transpose_block(sb[s], db[s], 0)
                store(i, s)
            return carry

        lax.fori_loop(0, per_w // 2, pair, 0, unroll=False)
        wait_store(per_w - 2, 0)
        wait_store(per_w - 1, 1)

        @pl.when(wid < extra)
        def _():
            i = per_w
            load(i, 0)
            wait_load(i, 0)
            transpose_block(sb[0], db[0], 0)
            store(i, 0)
            wait_store(i, 0)

        if tail:
            @pl.when(wid == NW - 1)
            def _():
                pltpu.sync_copy(tabT_hbm.at[:, pl.ds(V - tail, tail)], sbt)

                def tbody(k2, carry):
                    base = zeros16 + 2 * k2
                    for gj in range(2 * D // 16):
                        cols = base + (gj * 16) // D
                        db[1][k2, pl.ds(gj * 16, 16)] = plsc.load_gather(
                            sbt, [rows_l[gj], cols]
                        )
                    return carry

                lax.fori_loop(0, tail // 2, tbody, 0, unroll=4)
                pltpu.sync_copy(
                    db[1].at[pl.ds(0, tail // 2)],
                    out_hbm.at[pl.ds((V - tail) // 2, tail // 2)],
                )

    return tk


@functools.cache
def _gather(B, H, D):
    mesh = plsc.VectorSubcoreMesh(
        core_axis_name="c", subcore_axis_name="s", num_cores=NC, num_subcores=NS
    )
    nblk = BPT // C

    @functools.partial(
        pl.kernel,
        mesh=mesh,
        out_type=jax.ShapeDtypeStruct((H, D, B), jnp.float32),
        scratch_types=[
            pltpu.VMEM((H, BPT), jnp.int32),
            pltpu.VMEM((nblk, C), jnp.int32),
            pltpu.VMEM((nblk, C), jnp.int32),
            *[pltpu.VMEM((C, 2 * D), jnp.float32) for _ in range(nblk)],
            *[pltpu.VMEM((D, C), jnp.float32) for _ in range(2)],
            *[pltpu.SemaphoreType.DMA for _ in range(nblk)],
            *[pltpu.SemaphoreType.DMA for _ in range(2)],
        ],
        compiler_params=_params,
    )
    def gk(idxT_hbm, tabP_hbm, out_hbm, idx_v, vp_v, hf_v, *rest):
        bufs = rest[0:nblk]
        bufT = rest[nblk : nblk + 2]
        sem_g = rest[nblk + 2 : 2 * nblk + 2]
        sem_s = rest[2 * nblk + 2 : 2 * nblk + 4]
        wid = lax.axis_index("s") * NC + lax.axis_index("c")
        bs = wid * BPT
        pltpu.sync_copy(idxT_hbm.at[:, pl.ds(pl.multiple_of(bs, BPT), BPT)], idx_v)
        iota = lax.iota(jnp.int32, 16)

        def prep(h, blk):
            for g in range(C // 16):
                seg = idx_v[h, pl.ds(blk * C + g * 16, 16)]
                vp_v[blk, pl.ds(g * 16, 16)] = seg >> 1
                hf_v[blk, pl.ds(g * 16, 16)] = (seg & 1) * D

        def fire_gather(blk):
            pltpu.async_copy(tabP_hbm.at[vp_v.at[blk]], bufs[blk], sem_g[blk])

        def wait_gather(blk):
            pltpu.make_async_copy(
                tabP_hbm.at[vp_v.at[blk]], bufs[blk], sem_g[blk]
            ).wait()

        def fire_store(h, blk):
            pltpu.async_copy(
                bufT[blk % 2],
                out_hbm.at[h, :, pl.ds(pl.multiple_of(bs + blk * C, C), C)],
                sem_s[blk % 2],
            )

        def wait_store(h, blk):
            pltpu.make_async_copy(
                bufT[blk % 2],
                out_hbm.at[h, :, pl.ds(pl.multiple_of(bs + blk * C, C), C)],
                sem_s[blk % 2],
            ).wait()

        rows_l = tuple(iota + g * 16 for g in range(C // 16))

        def transpose_chunk(blk):
            src = bufs[blk]
            dst = bufT[blk % 2]
            hfs = tuple(hf_v[blk, pl.ds(g * 16, 16)] for g in range(C // 16))

            def body(d, carry):
                for g in range(C // 16):
                    dst[d, pl.ds(g * 16, 16)] = plsc.load_gather(
                        src, [rows_l[g], hfs[g] + d]
                    )
                return carry

            lax.fori_loop(0, D, body, 0, unroll=4)

        nch = H * nblk

        prep(0, 0)
        fire_gather(0)
        prep(0, 1)
        fire_gather(1)

        def do_h(h, carry):
            for blk in range(nblk):
                c = h * nblk + blk

                @pl.when(c + 2 < nch)
                def _():
                    h2 = h + (blk + 2) // nblk
                    blk2 = (blk + 2) % nblk
                    prep(h2, blk2)
                    fire_gather(blk2)

                wait_gather(blk)

                @pl.when(c >= 2)
                def _():
                    wait_store(h, blk)

                transpose_chunk(blk)
                fire_store(h, blk)
            return carry

        lax.fori_loop(0, H, do_h, 0, unroll=False)
        wait_store(H - 1, 2)
        wait_store(H - 1, 3)

    return gk


def kernel(input_variable, embedding_weight):
    B, H = input_variable.shape
    V, D = embedding_weight.shape
    idxT = input_variable.astype(jnp.int32).T
    tabT = embedding_weight.T
    tabP = _transpose(V, D)(tabT)
    out = _gather(B, H, D)(idxT, tabP)
    return jnp.transpose(out, (2, 0, 1))


# speedup vs baseline: 2.7329x; 2.1790x over previous
import functools

import jax
import jax.numpy as jnp
from jax import lax
from jax.experimental import pallas as pl
from jax.experimental.pallas import tpu as pltpu
from jax.experimental.pallas import tpu_sc as plsc

try:
    _info = plsc.get_sparse_core_info()
    NC, NS = _info.num_cores, _info.num_subcores
except Exception:
    NC, NS = 2, 16
NW = NC * NS

C = 128
S = 10
L = 5


@functools.cache
def _build(nchunk, D):
    mesh = plsc.VectorSubcoreMesh(
        core_axis_name="c", subcore_axis_name="s", num_cores=NC, num_subcores=NS
    )
    ngroups = nchunk // S

    @functools.partial(
        pl.kernel,
        mesh=mesh,
        out_type=jax.ShapeDtypeStruct((NW, nchunk, C, D), jnp.float32),
        scratch_types=[
            pltpu.VMEM((nchunk, C), jnp.int32),
            *[pltpu.VMEM((C, D), jnp.float32) for _ in range(S)],
            *[pltpu.SemaphoreType.DMA for _ in range(2 * S)],
        ],
        compiler_params=pltpu.CompilerParams(use_tc_tiling_on_sc=False),
    )
    def gather_kernel(idx_hbm, tab_hbm, out_hbm, idx_v, *rest):
        bufs = rest[:S]
        sem_g = rest[S : 2 * S]
        sem_s = rest[2 * S : 3 * S]
        wid = lax.axis_index("s") * NC + lax.axis_index("c")
        pltpu.sync_copy(idx_hbm.at[wid], idx_v)

        def fire_gather(j, b):
            pltpu.async_copy(tab_hbm.at[idx_v.at[j]], bufs[b], sem_g[b])

        def wait_gather(j, b):
            pltpu.make_async_copy(tab_hbm.at[idx_v.at[j]], bufs[b], sem_g[b]).wait()

        def fire_store(j, b):
            pltpu.async_copy(bufs[b], out_hbm.at[wid, j], sem_s[b])

        def wait_store(j, b):
            pltpu.make_async_copy(bufs[b], out_hbm.at[wid, j], sem_s[b]).wait()

        for b in range(L):
            fire_gather(b, b)

        for b in range(S):
            wait_gather(b, b)
            fire_store(b, b)
            jn, bn = b + L, (b + L) % S
            if jn >= S:
                wait_store(jn - S, bn)
            fire_gather(jn, bn)

        def group(g, carry):
            base = g * S
            for b in range(S):
                j = base + b
                wait_gather(j, b)
                fire_store(j, b)
                jn, bn = j + L, (b + L) % S
                wait_store(jn - S, bn)
                fire_gather(jn, bn)
            return carry

        lax.fori_loop(1, ngroups - 1, group, 0, unroll=False)

        base = (ngroups - 1) * S
        for b in range(S):
            j = base + b
            wait_gather(j, b)
            fire_store(j, b)
            jn, bn = j + L, (b + L) % S
            if jn < nchunk:
                wait_store(jn - S, bn)
                fire_gather(jn, bn)

        for b in range(S):
            wait_store(base + b, b)

    return gather_kernel


def kernel(input_variable, embedding_weight):
    B, H = input_variable.shape
    V, D = embedding_weight.shape
    total = B * H
    assert total % (NW * C) == 0
    nchunk = total // (NW * C)
    assert nchunk % S == 0 and nchunk // S >= 2
    idx = input_variable.reshape(NW, nchunk, C).astype(jnp.int32)
    out = _build(nchunk, D)(idx, embedding_weight)
    return out.reshape(B, H, D)
